# baseline (speedup 1.0000x reference)
import functools

import jax
import jax.numpy as jnp
from jax import lax
from jax.experimental import pallas as pl
from jax.experimental.pallas import tpu as pltpu
from jax.experimental.pallas import tpu_sc as plsc

B = 4096
BI = 1024
GAMMA = 0.9
MARGIN = 1.0

_SC_INFO = plsc.get_sparse_core_info()
_NC = _SC_INFO.num_cores
_NS = _SC_INFO.num_subcores
_TB = B // _NS


def _tc_body(yp_col_ref, yt_col_ref, idx_col_ref,
             s1_ref, s2_ref, m_ref, w_ref, npos_ref, nneg_ref,
             yjm_s, idxp_s, cnt_s, ypr_s, idxr_s):
    @pl.when(pl.program_id(0) == 0)
    def _():
        yj = yp_col_ref[...]
        ytj = yt_col_ref[...]
        idxj_all = idx_col_ref[...]
        yjm_s[...] = jnp.where(ytj == 0, yj, -1e30)
        idxp_s[...] = jnp.where(ytj == 1, idxj_all, -1)
        ypr_s[...] = yj.reshape(1, B)
        idxr_s[...] = idxj_all.reshape(1, B)
        npos = jnp.sum(jnp.where(ytj == 1, 1.0, 0.0))
        nneg = jnp.sum(jnp.where(ytj == 0, 1.0, 0.0))
        npos_ref[...] = jnp.full((1, 128), npos, jnp.float32)
        nneg_ref[...] = jnp.full((1, 128), nneg, jnp.float32)
        cnt_s[...] = jnp.full((1, 128), npos, jnp.float32)

    pid = pl.program_id(0)
    yjm = yjm_s[...]
    idxp = idxp_s[...]
    yi = ypr_s[0, pl.ds(pid * BI, BI)].reshape(1, BI)
    idxi = idxr_s[0, pl.ds(pid * BI, BI)].reshape(1, BI)
    npos = cnt_s[0, 0]

    yim = MARGIN - yi
    hn = jnp.maximum(yim + yjm, 0.0)
    hm = jnp.max(hn, axis=0, keepdims=True)
    msq = hm * hm
    m_ref[...] = msq
    hnsq = hn * hn
    e = jnp.exp(hnsq - msq)
    s1_ref[...] = (jnp.sum(e, axis=0, keepdims=True)
                   - npos * jnp.exp(-msq))
    s2_ref[...] = jnp.sum(e * hnsq, axis=0, keepdims=True)

    jiota = lax.broadcasted_iota(jnp.int32, (B, BI), 0)
    w = jnp.max(jnp.where(idxp == idxi, jiota, -1), axis=0, keepdims=True)
    w_ref[...] = jnp.maximum(w, 0)


def _tc_pass(yp_col, yt_col, idx_col):
    full = lambda i: (0, 0)
    blk = lambda i: (0, i)
    return pl.pallas_call(
        _tc_body,
        grid=(B // BI,),
        in_specs=[
            pl.BlockSpec((B, 1), full),
            pl.BlockSpec((B, 1), full),
            pl.BlockSpec((B, 1), full),
        ],
        out_specs=[pl.BlockSpec((1, BI), blk)] * 4 + [pl.BlockSpec((1, 128), full)] * 2,
        out_shape=[
            jax.ShapeDtypeStruct((1, B), jnp.float32),
            jax.ShapeDtypeStruct((1, B), jnp.float32),
            jax.ShapeDtypeStruct((1, B), jnp.float32),
            jax.ShapeDtypeStruct((1, B), jnp.int32),
            jax.ShapeDtypeStruct((1, 128), jnp.float32),
            jax.ShapeDtypeStruct((1, 128), jnp.float32),
        ],
        scratch_shapes=[
            pltpu.VMEM((B, 1), jnp.float32),
            pltpu.VMEM((B, 1), jnp.int32),
            pltpu.VMEM((1, 128), jnp.float32),
            pltpu.VMEM((1, B), jnp.float32),
            pltpu.VMEM((1, B), jnp.int32),
        ],
    )(yp_col, yt_col, idx_col)


_MESH = plsc.VectorSubcoreMesh(core_axis_name="c", subcore_axis_name="s", num_cores=1)


@functools.partial(
    pl.kernel,
    mesh=_MESH,
    out_type=jax.ShapeDtypeStruct((16,), jnp.float32),
    scratch_types=[
        pltpu.VMEM((_TB,), jnp.int32),
        pltpu.VMEM((_TB,), jnp.int32),
        pltpu.VMEM((_TB,), jnp.int32),
        pltpu.VMEM((_TB,), jnp.float32),
        pltpu.VMEM((_TB,), jnp.float32),
        pltpu.VMEM((_TB,), jnp.float32),
        pltpu.VMEM((_TB,), jnp.float32),
        pltpu.VMEM((_TB,), jnp.float32),
        pltpu.VMEM((128,), jnp.float32),
        pltpu.VMEM((128,), jnp.float32),
        pltpu.VMEM((16,), jnp.float32),
        pltpu.VMEM((16,), jnp.int32),
        pltpu.VMEM((16,), jnp.float32),
        pltpu.VMEM((16,), jnp.float32),
        pltpu.VMEM_SHARED((16,), jnp.float32),
        pltpu.SemaphoreType.DMA,
    ],
)
def _sc_combine(u_hbm, idx_hbm, w_hbm, yt_hbm, s1_hbm, s2_hbm, m_hbm,
                npos_hbm, nneg_hbm, out_hbm,
                idx_v, wv_v, yt_v, s2_v, m_v, g_v, s1w_v, mw_v,
                np_v, nn_v, acc_v, zid_v, zf_v, out_v, red_sh, sem):
    cid = lax.axis_index("c")
    sid = lax.axis_index("s")
    wid = sid * _NC + cid

    @pl.when(wid == 0)
    def _():
        zf_v[...] = jnp.zeros((16,), jnp.float32)
        pltpu.sync_copy(zf_v, red_sh)

    plsc.subcore_barrier()

    @pl.when(cid == 0)
    def _():
        base = sid * _TB
        sl_own = pl.ds(base, _TB)
        pltpu.sync_copy(idx_hbm.at[sl_own], idx_v)
        pltpu.sync_copy(w_hbm.at[sl_own], wv_v)
        lo, hi = pl.ds(0, 128), pl.ds(128, 128)
        h1 = pltpu.async_copy(u_hbm.at[idx_v.at[lo]], g_v.at[lo], sem)
        h2 = pltpu.async_copy(u_hbm.at[idx_v.at[hi]], g_v.at[hi], sem)
        h3 = pltpu.async_copy(s1_hbm.at[wv_v.at[lo]], s1w_v.at[lo], sem)
        h4 = pltpu.async_copy(s1_hbm.at[wv_v.at[hi]], s1w_v.at[hi], sem)
        h5 = pltpu.async_copy(m_hbm.at[wv_v.at[lo]], mw_v.at[lo], sem)
        h6 = pltpu.async_copy(m_hbm.at[wv_v.at[hi]], mw_v.at[hi], sem)
        pltpu.sync_copy(yt_hbm.at[sl_own], yt_v)
        pltpu.sync_copy(s2_hbm.at[sl_own], s2_v)
        pltpu.sync_copy(m_hbm.at[sl_own], m_v)
        pltpu.sync_copy(npos_hbm, np_v)
        pltpu.sync_copy(nneg_hbm, nn_v)
        h1.wait(); h2.wait(); h3.wait(); h4.wait(); h5.wait(); h6.wait()

        coef = GAMMA / nn_v[pl.ds(0, 16)]

        def loss_body(k, acc):
            sl = pl.ds(k * 16, 16)
            denom = ((1.0 - GAMMA) * g_v[sl] * jnp.exp(-m_v[sl])
                     + coef * s1w_v[sl] * jnp.exp(mw_v[sl] - m_v[sl]))
            term = s2_v[sl] / denom
            return acc + jnp.where(yt_v[sl] == 1, term, 0.0)

        acc = lax.fori_loop(0, _TB // 16, loss_body,
                            jnp.zeros((16,), jnp.float32))
        acc_v[...] = acc
        zid_v[...] = jnp.zeros((16,), jnp.int32)
        pltpu.sync_copy(acc_v, red_sh.at[zid_v], add=True)

    plsc.subcore_barrier()

    @pl.when(wid == 0)
    def _():
        pltpu.sync_copy(red_sh, zf_v)
        npos = np_v[pl.ds(0, 16)]
        nneg = nn_v[pl.ds(0, 16)]
        out_v[...] = zf_v[...] / (npos * nneg)
        pltpu.sync_copy(out_v, out_hbm)


def kernel(y_pred, y_true, index, u_pos):
    yp = y_pred.reshape(B).astype(jnp.float32)
    yt = y_true.reshape(B).astype(jnp.int32)
    idx = index.reshape(B).astype(jnp.int32)
    up = u_pos.reshape(-1)

    s1, s2, m, w, npos, nneg = _tc_pass(
        yp.reshape(B, 1), yt.reshape(B, 1), idx.reshape(B, 1))

    out16 = _sc_combine(up, idx, w.reshape(B), yt, s1.reshape(B),
                        s2.reshape(B), m.reshape(B),
                        npos.reshape(128), nneg.reshape(128))
    return out16[0]


# speedup vs baseline: 1.1029x; 1.1029x over previous
import functools

import jax
import jax.numpy as jnp
from jax import lax
from jax.experimental import pallas as pl
from jax.experimental.pallas import tpu as pltpu
from jax.experimental.pallas import tpu_sc as plsc

B = 4096
BI = 1024
GAMMA = 0.9
MARGIN = 1.0

_SC_INFO = plsc.get_sparse_core_info()
_NC = _SC_INFO.num_cores
_NS = _SC_INFO.num_subcores
_TB = B // _NS


def _tc_body(yp_col_ref, yt_col_ref, idx_col_ref, yp_row_ref, idx_row_ref,
             yt_row_ref,
             s1_ref, s2_ref, m_ref, w_ref, npos_ref, nneg_ref,
             yjm_s, idxp_s, cnt_s, mneg_s):
    @pl.when(pl.program_id(0) == 0)
    def _():
        yj = yp_col_ref[...]
        ytj = yt_col_ref[...]
        yjm = jnp.where(ytj == 0, yj, -1e30)
        yjm_s[...] = yjm
        idxp_s[...] = jnp.where(ytj == 1, idx_col_ref[...], -1)
        ytr = yt_row_ref[...]
        npos = jnp.sum(jnp.where(ytr == 1, 1.0, 0.0))
        nneg = jnp.sum(jnp.where(ytr == 0, 1.0, 0.0))
        npos_ref[...] = jnp.full((1, 128), npos, jnp.float32)
        nneg_ref[...] = jnp.full((1, 128), nneg, jnp.float32)
        cnt_s[...] = jnp.full((1, 128), npos, jnp.float32)
        mneg_s[...] = jnp.full((1, 128), jnp.max(yjm), jnp.float32)

    yjm = yjm_s[...]
    idxp = idxp_s[...]
    yi = yp_row_ref[...]
    idxi = idx_row_ref[...]
    npos = cnt_s[0, 0]
    maxneg = mneg_s[0, 0]

    yim = MARGIN - yi
    hm = jnp.maximum(yim + maxneg, 0.0)
    msq = hm * hm
    m_ref[...] = msq
    hn = jnp.maximum(yim + yjm, 0.0)
    hnsq = hn * hn
    e = jnp.exp(hnsq - msq)
    s1_ref[...] = (jnp.sum(e, axis=0, keepdims=True)
                   - npos * jnp.exp(-msq))
    s2_ref[...] = jnp.sum(e * hnsq, axis=0, keepdims=True)

    jiota = lax.broadcasted_iota(jnp.int32, (B, BI), 0)
    w = jnp.max(jnp.where(idxp == idxi, jiota, -1), axis=0, keepdims=True)
    w_ref[...] = jnp.maximum(w, 0)


def _tc_pass(yp_col, yt_col, idx_col, yp_row, idx_row, yt_row):
    full = lambda i: (0, 0)
    blk = lambda i: (0, i)
    return pl.pallas_call(
        _tc_body,
        grid=(B // BI,),
        in_specs=[
            pl.BlockSpec((B, 1), full),
            pl.BlockSpec((B, 1), full),
            pl.BlockSpec((B, 1), full),
            pl.BlockSpec((1, BI), blk),
            pl.BlockSpec((1, BI), blk),
            pl.BlockSpec((1, B), full),
        ],
        out_specs=[pl.BlockSpec((1, BI), blk)] * 4 + [pl.BlockSpec((1, 128), full)] * 2,
        out_shape=[
            jax.ShapeDtypeStruct((1, B), jnp.float32),
            jax.ShapeDtypeStruct((1, B), jnp.float32),
            jax.ShapeDtypeStruct((1, B), jnp.float32),
            jax.ShapeDtypeStruct((1, B), jnp.int32),
            jax.ShapeDtypeStruct((1, 128), jnp.float32),
            jax.ShapeDtypeStruct((1, 128), jnp.float32),
        ],
        scratch_shapes=[
            pltpu.VMEM((B, 1), jnp.float32),
            pltpu.VMEM((B, 1), jnp.int32),
            pltpu.VMEM((1, 128), jnp.float32),
            pltpu.VMEM((1, 128), jnp.float32),
        ],
    )(yp_col, yt_col, idx_col, yp_row, idx_row, yt_row)


_MESH = plsc.VectorSubcoreMesh(core_axis_name="c", subcore_axis_name="s", num_cores=1)


@functools.partial(
    pl.kernel,
    mesh=_MESH,
    out_type=jax.ShapeDtypeStruct((16,), jnp.float32),
    scratch_types=[
        pltpu.VMEM((_TB,), jnp.int32),
        pltpu.VMEM((_TB,), jnp.int32),
        pltpu.VMEM((_TB,), jnp.int32),
        pltpu.VMEM((_TB,), jnp.float32),
        pltpu.VMEM((_TB,), jnp.float32),
        pltpu.VMEM((_TB,), jnp.float32),
        pltpu.VMEM((_TB,), jnp.float32),
        pltpu.VMEM((_TB,), jnp.float32),
        pltpu.VMEM((128,), jnp.float32),
        pltpu.VMEM((128,), jnp.float32),
        pltpu.VMEM((16,), jnp.float32),
        pltpu.VMEM((16,), jnp.int32),
        pltpu.VMEM((16,), jnp.float32),
        pltpu.VMEM((16,), jnp.float32),
        pltpu.VMEM_SHARED((16,), jnp.float32),
        pltpu.SemaphoreType.DMA,
    ],
)
def _sc_combine(u_hbm, idx_hbm, w_hbm, yt_hbm, s1_hbm, s2_hbm, m_hbm,
                npos_hbm, nneg_hbm, out_hbm,
                idx_v, wv_v, yt_v, s2_v, m_v, g_v, s1w_v, mw_v,
                np_v, nn_v, acc_v, zid_v, zf_v, out_v, red_sh, sem):
    cid = lax.axis_index("c")
    sid = lax.axis_index("s")
    wid = sid * _NC + cid

    @pl.when(wid == 0)
    def _():
        zf_v[...] = jnp.zeros((16,), jnp.float32)
        pltpu.sync_copy(zf_v, red_sh)

    plsc.subcore_barrier()

    @pl.when(cid == 0)
    def _():
        base = sid * _TB
        sl_own = pl.ds(base, _TB)
        pltpu.sync_copy(idx_hbm.at[sl_own], idx_v)
        pltpu.sync_copy(w_hbm.at[sl_own], wv_v)
        lo, hi = pl.ds(0, 128), pl.ds(128, 128)
        h1 = pltpu.async_copy(u_hbm.at[idx_v.at[lo]], g_v.at[lo], sem)
        h2 = pltpu.async_copy(u_hbm.at[idx_v.at[hi]], g_v.at[hi], sem)
        h3 = pltpu.async_copy(s1_hbm.at[wv_v.at[lo]], s1w_v.at[lo], sem)
        h4 = pltpu.async_copy(s1_hbm.at[wv_v.at[hi]], s1w_v.at[hi], sem)
        h5 = pltpu.async_copy(m_hbm.at[wv_v.at[lo]], mw_v.at[lo], sem)
        h6 = pltpu.async_copy(m_hbm.at[wv_v.at[hi]], mw_v.at[hi], sem)
        pltpu.sync_copy(yt_hbm.at[sl_own], yt_v)
        pltpu.sync_copy(s2_hbm.at[sl_own], s2_v)
        pltpu.sync_copy(m_hbm.at[sl_own], m_v)
        pltpu.sync_copy(npos_hbm, np_v)
        pltpu.sync_copy(nneg_hbm, nn_v)
        h1.wait(); h2.wait(); h3.wait(); h4.wait(); h5.wait(); h6.wait()

        coef = GAMMA / nn_v[pl.ds(0, 16)]

        def loss_body(k, acc):
            sl = pl.ds(k * 16, 16)
            denom = ((1.0 - GAMMA) * g_v[sl] * jnp.exp(-m_v[sl])
                     + coef * s1w_v[sl] * jnp.exp(mw_v[sl] - m_v[sl]))
            term = s2_v[sl] / denom
            return acc + jnp.where(yt_v[sl] == 1, term, 0.0)

        acc = lax.fori_loop(0, _TB // 16, loss_body,
                            jnp.zeros((16,), jnp.float32))
        acc_v[...] = acc
        zid_v[...] = jnp.zeros((16,), jnp.int32)
        pltpu.sync_copy(acc_v, red_sh.at[zid_v], add=True)

    plsc.subcore_barrier()

    @pl.when(wid == 0)
    def _():
        pltpu.sync_copy(red_sh, zf_v)
        npos = np_v[pl.ds(0, 16)]
        nneg = nn_v[pl.ds(0, 16)]
        out_v[...] = zf_v[...] / (npos * nneg)
        pltpu.sync_copy(out_v, out_hbm)


def kernel(y_pred, y_true, index, u_pos):
    yp = y_pred.reshape(B).astype(jnp.float32)
    yt = y_true.reshape(B).astype(jnp.int32)
    idx = index.reshape(B).astype(jnp.int32)
    up = u_pos.reshape(-1)

    s1, s2, m, w, npos, nneg = _tc_pass(
        yp.reshape(B, 1), yt.reshape(B, 1), idx.reshape(B, 1),
        yp.reshape(1, B), idx.reshape(1, B), yt.reshape(1, B))

    out16 = _sc_combine(up, idx, w.reshape(B), yt, s1.reshape(B),
                        s2.reshape(B), m.reshape(B),
                        npos.reshape(128), nneg.reshape(128))
    return out16[0]


# speedup vs baseline: 1.1300x; 1.0246x over previous
import functools

import jax
import jax.numpy as jnp
from jax import lax
from jax.experimental import pallas as pl
from jax.experimental.pallas import tpu as pltpu
from jax.experimental.pallas import tpu_sc as plsc

B = 4096
BI = 1024
GAMMA = 0.9
MARGIN = 1.0

_SC_INFO = plsc.get_sparse_core_info()
_NC = _SC_INFO.num_cores
_NS = _SC_INFO.num_subcores
_TB = B // _NS


def _tc_body(yp_col_ref, yt_col_ref, idx_col_ref, yp_row_ref, idx_row_ref,
             yt_row_ref,
             s1_ref, s2_ref, m_ref, w_ref, npos_ref, nneg_ref,
             yjm_s, idxp_s, cnt_s, mneg_s):
    @pl.when(pl.program_id(0) == 0)
    def _():
        yj = yp_col_ref[...]
        ytj = yt_col_ref[...]
        yjm = jnp.where(ytj == 0, yj, -1e30)
        yjm_s[...] = yjm
        idxp_s[...] = jnp.where(ytj == 1, idx_col_ref[...], -1)
        ytr = yt_row_ref[...]
        npos = jnp.sum(jnp.where(ytr == 1, 1.0, 0.0))
        nneg = jnp.sum(jnp.where(ytr == 0, 1.0, 0.0))
        npos_ref[...] = jnp.full((1, 128), npos, jnp.float32)
        nneg_ref[...] = jnp.full((1, 128), nneg, jnp.float32)
        cnt_s[...] = jnp.full((1, 128), npos, jnp.float32)
        mneg_s[...] = jnp.full((1, 128), jnp.max(yjm), jnp.float32)

    yjm = yjm_s[...]
    idxp = idxp_s[...]
    yi = yp_row_ref[...]
    idxi = idx_row_ref[...]
    npos = cnt_s[0, 0]
    maxneg = mneg_s[0, 0]

    yim = MARGIN - yi
    hm = jnp.maximum(yim + maxneg, 0.0)
    msq = hm * hm
    m_ref[...] = msq
    hn = jnp.maximum(yim + yjm, 0.0)
    hnsq = hn * hn
    e = jnp.exp(hnsq - msq)
    s1_ref[...] = (jnp.sum(e, axis=0, keepdims=True)
                   - npos * jnp.exp(-msq))
    s2_ref[...] = jnp.sum(e * hnsq, axis=0, keepdims=True)

    jiota = lax.broadcasted_iota(jnp.int32, (B, BI), 0).astype(jnp.float32)
    wf = jnp.max(jnp.where(idxp == idxi, jiota, 0.0), axis=0, keepdims=True)
    w_ref[...] = wf.astype(jnp.int32)


def _tc_pass(yp_col, yt_col, idx_col, yp_row, idx_row, yt_row):
    full = lambda i: (0, 0)
    blk = lambda i: (0, i)
    return pl.pallas_call(
        _tc_body,
        grid=(B // BI,),
        in_specs=[
            pl.BlockSpec((B, 1), full),
            pl.BlockSpec((B, 1), full),
            pl.BlockSpec((B, 1), full),
            pl.BlockSpec((1, BI), blk),
            pl.BlockSpec((1, BI), blk),
            pl.BlockSpec((1, B), full),
        ],
        out_specs=[pl.BlockSpec((1, BI), blk)] * 4 + [pl.BlockSpec((1, 128), full)] * 2,
        out_shape=[
            jax.ShapeDtypeStruct((1, B), jnp.float32),
            jax.ShapeDtypeStruct((1, B), jnp.float32),
            jax.ShapeDtypeStruct((1, B), jnp.float32),
            jax.ShapeDtypeStruct((1, B), jnp.int32),
            jax.ShapeDtypeStruct((1, 128), jnp.float32),
            jax.ShapeDtypeStruct((1, 128), jnp.float32),
        ],
        scratch_shapes=[
            pltpu.VMEM((B, 1), jnp.float32),
            pltpu.VMEM((B, 1), jnp.int32),
            pltpu.VMEM((1, 128), jnp.float32),
            pltpu.VMEM((1, 128), jnp.float32),
        ],
    )(yp_col, yt_col, idx_col, yp_row, idx_row, yt_row)


_MESH = plsc.VectorSubcoreMesh(core_axis_name="c", subcore_axis_name="s", num_cores=1)


@functools.partial(
    pl.kernel,
    mesh=_MESH,
    out_type=jax.ShapeDtypeStruct((16,), jnp.float32),
    scratch_types=[
        pltpu.VMEM((_TB,), jnp.int32),
        pltpu.VMEM((_TB,), jnp.int32),
        pltpu.VMEM((_TB,), jnp.int32),
        pltpu.VMEM((_TB,), jnp.float32),
        pltpu.VMEM((_TB,), jnp.float32),
        pltpu.VMEM((_TB,), jnp.float32),
        pltpu.VMEM((_TB,), jnp.float32),
        pltpu.VMEM((_TB,), jnp.float32),
        pltpu.VMEM((128,), jnp.float32),
        pltpu.VMEM((128,), jnp.float32),
        pltpu.VMEM((16,), jnp.float32),
        pltpu.VMEM((16,), jnp.int32),
        pltpu.VMEM((16,), jnp.float32),
        pltpu.VMEM((16,), jnp.float32),
        pltpu.VMEM_SHARED((16,), jnp.float32),
        pltpu.SemaphoreType.DMA,
    ],
)
def _sc_combine(u_hbm, idx_hbm, w_hbm, yt_hbm, s1_hbm, s2_hbm, m_hbm,
                npos_hbm, nneg_hbm, out_hbm,
                idx_v, wv_v, yt_v, s2_v, m_v, g_v, s1w_v, mw_v,
                np_v, nn_v, acc_v, zid_v, zf_v, out_v, red_sh, sem):
    cid = lax.axis_index("c")
    sid = lax.axis_index("s")
    wid = sid * _NC + cid

    @pl.when(wid == 0)
    def _():
        zf_v[...] = jnp.zeros((16,), jnp.float32)
        pltpu.sync_copy(zf_v, red_sh)

    plsc.subcore_barrier()

    @pl.when(cid == 0)
    def _():
        base = sid * _TB
        sl_own = pl.ds(base, _TB)
        pltpu.sync_copy(idx_hbm.at[sl_own], idx_v)
        pltpu.sync_copy(w_hbm.at[sl_own], wv_v)
        lo, hi = pl.ds(0, 128), pl.ds(128, 128)
        h1 = pltpu.async_copy(u_hbm.at[idx_v.at[lo]], g_v.at[lo], sem)
        h2 = pltpu.async_copy(u_hbm.at[idx_v.at[hi]], g_v.at[hi], sem)
        h3 = pltpu.async_copy(s1_hbm.at[wv_v.at[lo]], s1w_v.at[lo], sem)
        h4 = pltpu.async_copy(s1_hbm.at[wv_v.at[hi]], s1w_v.at[hi], sem)
        h5 = pltpu.async_copy(m_hbm.at[wv_v.at[lo]], mw_v.at[lo], sem)
        h6 = pltpu.async_copy(m_hbm.at[wv_v.at[hi]], mw_v.at[hi], sem)
        pltpu.sync_copy(yt_hbm.at[sl_own], yt_v)
        pltpu.sync_copy(s2_hbm.at[sl_own], s2_v)
        pltpu.sync_copy(m_hbm.at[sl_own], m_v)
        pltpu.sync_copy(npos_hbm, np_v)
        pltpu.sync_copy(nneg_hbm, nn_v)
        h1.wait(); h2.wait(); h3.wait(); h4.wait(); h5.wait(); h6.wait()

        coef = GAMMA / nn_v[pl.ds(0, 16)]

        def loss_body(k, acc):
            sl = pl.ds(k * 16, 16)
            denom = ((1.0 - GAMMA) * g_v[sl] * jnp.exp(-m_v[sl])
                     + coef * s1w_v[sl] * jnp.exp(mw_v[sl] - m_v[sl]))
            term = s2_v[sl] / denom
            return acc + jnp.where(yt_v[sl] == 1, term, 0.0)

        acc = lax.fori_loop(0, _TB // 16, loss_body,
                            jnp.zeros((16,), jnp.float32))
        acc_v[...] = acc
        zid_v[...] = jnp.zeros((16,), jnp.int32)
        pltpu.sync_copy(acc_v, red_sh.at[zid_v], add=True)

    plsc.subcore_barrier()

    @pl.when(wid == 0)
    def _():
        pltpu.sync_copy(red_sh, zf_v)
        npos = np_v[pl.ds(0, 16)]
        nneg = nn_v[pl.ds(0, 16)]
        out_v[...] = zf_v[...] / (npos * nneg)
        pltpu.sync_copy(out_v, out_hbm)


def kernel(y_pred, y_true, index, u_pos):
    yp = y_pred.reshape(B).astype(jnp.float32)
    yt = y_true.reshape(B).astype(jnp.int32)
    idx = index.reshape(B).astype(jnp.int32)
    up = u_pos.reshape(-1)

    s1, s2, m, w, npos, nneg = _tc_pass(
        yp.reshape(B, 1), yt.reshape(B, 1), idx.reshape(B, 1),
        yp.reshape(1, B), idx.reshape(1, B), yt.reshape(1, B))

    out16 = _sc_combine(up, idx, w.reshape(B), yt, s1.reshape(B),
                        s2.reshape(B), m.reshape(B),
                        npos.reshape(128), nneg.reshape(128))
    return out16[0]


# speedup vs baseline: 1.1514x; 1.0190x over previous
import functools

import jax
import jax.numpy as jnp
from jax import lax
from jax.experimental import pallas as pl
from jax.experimental.pallas import tpu as pltpu
from jax.experimental.pallas import tpu_sc as plsc

B = 4096
BI = 1024
GAMMA = 0.9
MARGIN = 1.0

_SC_INFO = plsc.get_sparse_core_info()
_NC = _SC_INFO.num_cores
_NS = _SC_INFO.num_subcores
_TB = B // _NS


def _tc_body(yp_col_ref, yt_col_ref, idx_col_ref, yp_row_ref, idx_row_ref,
             yt_row_ref,
             s1_ref, s2_ref, m_ref, w_ref, npos_ref, nneg_ref,
             yjm_s, idxp_s, cnt_s, mneg_s):
    @pl.when(pl.program_id(0) == 0)
    def _():
        yj = yp_col_ref[...]
        ytj = yt_col_ref[...]
        yjm = jnp.where(ytj == 0, yj, -1e30)
        yjm_s[...] = yjm * jnp.float32(1.2011224087864498)
        idxp_s[...] = jnp.where(ytj == 1, idx_col_ref[...], -1)
        ytr = yt_row_ref[...]
        npos = jnp.sum(jnp.where(ytr == 1, 1.0, 0.0))
        nneg = jnp.sum(jnp.where(ytr == 0, 1.0, 0.0))
        npos_ref[...] = jnp.full((1, 128), npos, jnp.float32)
        nneg_ref[...] = jnp.full((1, 128), nneg, jnp.float32)
        cnt_s[...] = jnp.full((1, 128), npos, jnp.float32)
        mneg_s[...] = jnp.full((1, 128), jnp.max(yjm), jnp.float32)

    yjm = yjm_s[...]
    idxp = idxp_s[...]
    yi = yp_row_ref[...]
    idxi = idx_row_ref[...]
    npos = cnt_s[0, 0]
    maxneg = mneg_s[0, 0]

    RT = jnp.float32(1.2011224087864498)
    L2E = jnp.float32(1.4426950408889634)
    yim = MARGIN - yi
    hm = jnp.maximum(yim + maxneg, 0.0)
    msq = hm * hm
    m_ref[...] = msq
    msqL = msq * L2E
    yimL = yim * RT
    hnL = jnp.maximum(yimL + yjm, 0.0)
    hnsqL = hnL * hnL
    e = jnp.exp2(hnsqL - msqL)
    s1_ref[...] = (jnp.sum(e, axis=0, keepdims=True)
                   - npos * jnp.exp(-msq))
    s2_ref[...] = jnp.sum(e * hnsqL, axis=0, keepdims=True) * (1.0 / L2E)

    jiota = lax.broadcasted_iota(jnp.int32, (B, BI), 0).astype(jnp.float32)
    wf = jnp.max(jnp.where(idxp == idxi, jiota, 0.0), axis=0, keepdims=True)
    w_ref[...] = wf.astype(jnp.int32)


def _tc_pass(yp_col, yt_col, idx_col, yp_row, idx_row, yt_row):
    full = lambda i: (0, 0)
    blk = lambda i: (0, i)
    return pl.pallas_call(
        _tc_body,
        grid=(B // BI,),
        in_specs=[
            pl.BlockSpec((B, 1), full),
            pl.BlockSpec((B, 1), full),
            pl.BlockSpec((B, 1), full),
            pl.BlockSpec((1, BI), blk),
            pl.BlockSpec((1, BI), blk),
            pl.BlockSpec((1, B), full),
        ],
        out_specs=[pl.BlockSpec((1, BI), blk)] * 4 + [pl.BlockSpec((1, 128), full)] * 2,
        out_shape=[
            jax.ShapeDtypeStruct((1, B), jnp.float32),
            jax.ShapeDtypeStruct((1, B), jnp.float32),
            jax.ShapeDtypeStruct((1, B), jnp.float32),
            jax.ShapeDtypeStruct((1, B), jnp.int32),
            jax.ShapeDtypeStruct((1, 128), jnp.float32),
            jax.ShapeDtypeStruct((1, 128), jnp.float32),
        ],
        scratch_shapes=[
            pltpu.VMEM((B, 1), jnp.float32),
            pltpu.VMEM((B, 1), jnp.int32),
            pltpu.VMEM((1, 128), jnp.float32),
            pltpu.VMEM((1, 128), jnp.float32),
        ],
    )(yp_col, yt_col, idx_col, yp_row, idx_row, yt_row)


_MESH = plsc.VectorSubcoreMesh(core_axis_name="c", subcore_axis_name="s", num_cores=1)


@functools.partial(
    pl.kernel,
    mesh=_MESH,
    out_type=jax.ShapeDtypeStruct((16,), jnp.float32),
    scratch_types=[
        pltpu.VMEM((_TB,), jnp.int32),
        pltpu.VMEM((_TB,), jnp.int32),
        pltpu.VMEM((_TB,), jnp.int32),
        pltpu.VMEM((_TB,), jnp.float32),
        pltpu.VMEM((_TB,), jnp.float32),
        pltpu.VMEM((_TB,), jnp.float32),
        pltpu.VMEM((_TB,), jnp.float32),
        pltpu.VMEM((_TB,), jnp.float32),
        pltpu.VMEM((128,), jnp.float32),
        pltpu.VMEM((128,), jnp.float32),
        pltpu.VMEM((16,), jnp.float32),
        pltpu.VMEM((16,), jnp.int32),
        pltpu.VMEM((16,), jnp.float32),
        pltpu.VMEM((16,), jnp.float32),
        pltpu.VMEM_SHARED((16,), jnp.float32),
        pltpu.SemaphoreType.DMA,
    ],
)
def _sc_combine(u_hbm, idx_hbm, w_hbm, yt_hbm, s1_hbm, s2_hbm, m_hbm,
                npos_hbm, nneg_hbm, out_hbm,
                idx_v, wv_v, yt_v, s2_v, m_v, g_v, s1w_v, mw_v,
                np_v, nn_v, acc_v, zid_v, zf_v, out_v, red_sh, sem):
    cid = lax.axis_index("c")
    sid = lax.axis_index("s")
    wid = sid * _NC + cid

    @pl.when(wid == 0)
    def _():
        zf_v[...] = jnp.zeros((16,), jnp.float32)
        pltpu.sync_copy(zf_v, red_sh)

    plsc.subcore_barrier()

    @pl.when(cid == 0)
    def _():
        base = sid * _TB
        sl_own = pl.ds(base, _TB)
        pltpu.sync_copy(idx_hbm.at[sl_own], idx_v)
        pltpu.sync_copy(w_hbm.at[sl_own], wv_v)
        lo, hi = pl.ds(0, 128), pl.ds(128, 128)
        h1 = pltpu.async_copy(u_hbm.at[idx_v.at[lo]], g_v.at[lo], sem)
        h2 = pltpu.async_copy(u_hbm.at[idx_v.at[hi]], g_v.at[hi], sem)
        h3 = pltpu.async_copy(s1_hbm.at[wv_v.at[lo]], s1w_v.at[lo], sem)
        h4 = pltpu.async_copy(s1_hbm.at[wv_v.at[hi]], s1w_v.at[hi], sem)
        h5 = pltpu.async_copy(m_hbm.at[wv_v.at[lo]], mw_v.at[lo], sem)
        h6 = pltpu.async_copy(m_hbm.at[wv_v.at[hi]], mw_v.at[hi], sem)
        pltpu.sync_copy(yt_hbm.at[sl_own], yt_v)
        pltpu.sync_copy(s2_hbm.at[sl_own], s2_v)
        pltpu.sync_copy(m_hbm.at[sl_own], m_v)
        pltpu.sync_copy(npos_hbm, np_v)
        pltpu.sync_copy(nneg_hbm, nn_v)
        h1.wait(); h2.wait(); h3.wait(); h4.wait(); h5.wait(); h6.wait()

        coef = GAMMA / nn_v[pl.ds(0, 16)]

        def loss_body(k, acc):
            sl = pl.ds(k * 16, 16)
            denom = ((1.0 - GAMMA) * g_v[sl] * jnp.exp(-m_v[sl])
                     + coef * s1w_v[sl] * jnp.exp(mw_v[sl] - m_v[sl]))
            term = s2_v[sl] / denom
            return acc + jnp.where(yt_v[sl] == 1, term, 0.0)

        acc = lax.fori_loop(0, _TB // 16, loss_body,
                            jnp.zeros((16,), jnp.float32))
        acc_v[...] = acc
        zid_v[...] = jnp.zeros((16,), jnp.int32)
        pltpu.sync_copy(acc_v, red_sh.at[zid_v], add=True)

    plsc.subcore_barrier()

    @pl.when(wid == 0)
    def _():
        pltpu.sync_copy(red_sh, zf_v)
        npos = np_v[pl.ds(0, 16)]
        nneg = nn_v[pl.ds(0, 16)]
        out_v[...] = zf_v[...] / (npos * nneg)
        pltpu.sync_copy(out_v, out_hbm)


def kernel(y_pred, y_true, index, u_pos):
    yp = y_pred.reshape(B).astype(jnp.float32)
    yt = y_true.reshape(B).astype(jnp.int32)
    idx = index.reshape(B).astype(jnp.int32)
    up = u_pos.reshape(-1)

    s1, s2, m, w, npos, nneg = _tc_pass(
        yp.reshape(B, 1), yt.reshape(B, 1), idx.reshape(B, 1),
        yp.reshape(1, B), idx.reshape(1, B), yt.reshape(1, B))

    out16 = _sc_combine(up, idx, w.reshape(B), yt, s1.reshape(B),
                        s2.reshape(B), m.reshape(B),
                        npos.reshape(128), nneg.reshape(128))
    return out16[0]
